# baseline (speedup 1.0000x reference)
import numpy as np
import jax
import jax.numpy as jnp
from jax import lax
from functools import partial
from jax.experimental import pallas as pl
from jax.experimental.pallas import tpu as pltpu

_TEMPERATURE = 0.12
_MAX_STEP_SIZE = 0.18
_MOMENTUM = 0.45
_EXPLORATION_RATE = 0.45

_R1 = (13, 15, 26, 6)
_R2 = (17, 29, 16, 24)
_TINY = np.float32(np.finfo(np.float32).tiny)

_KS_DATA = np.asarray(
    jax.random.key_data(jax.random.split(jax.random.key(42), 4)[1]),
    dtype=np.uint32,
)


def _threefry2x32(k0, k1, x0, x1):
    ks0 = np.uint32(k0)
    ks1 = np.uint32(k1)
    ks2 = np.uint32((int(k0) ^ int(k1) ^ 0x1BD11BDA) & 0xFFFFFFFF)

    def rnds(x0, x1, rots):
        for r in rots:
            x0 = x0 + x1
            x1 = (x1 << np.uint32(r)) | (x1 >> np.uint32(32 - r))
            x1 = x1 ^ x0
        return x0, x1

    x0 = x0 + ks0
    x1 = x1 + ks1
    x0, x1 = rnds(x0, x1, _R1)
    x0 = x0 + ks1
    x1 = x1 + np.uint32((int(ks2) + 1) & 0xFFFFFFFF)
    x0, x1 = rnds(x0, x1, _R2)
    x0 = x0 + ks2
    x1 = x1 + np.uint32((int(ks0) + 2) & 0xFFFFFFFF)
    x0, x1 = rnds(x0, x1, _R1)
    x0 = x0 + ks0
    x1 = x1 + np.uint32((int(ks1) + 3) & 0xFFFFFFFF)
    x0, x1 = rnds(x0, x1, _R2)
    x0 = x0 + ks1
    x1 = x1 + np.uint32((int(ks2) + 4) & 0xFFFFFFFF)
    x0, x1 = rnds(x0, x1, _R1)
    x0 = x0 + ks2
    x1 = x1 + np.uint32((int(ks0) + 5) & 0xFFFFFFFF)
    return x0, x1


def _sample_kernel(sal_ref, out_ref, *, k0, k1, H, W, CH):
    b = pl.program_id(0)
    n_chunks = H // CH
    base = jnp.uint32(H * W) * jnp.uint32(b)
    jrow = lax.broadcasted_iota(jnp.uint32, (CH, W), 0) * jnp.uint32(W)
    jcol = lax.broadcasted_iota(jnp.uint32, (CH, W), 1)
    jloc = jrow + jcol

    def body(i, carry):
        m, idx = carry
        sal = sal_ref[0, pl.ds(i * CH, CH), :]
        j = jnp.uint32(CH * W) * i.astype(jnp.uint32) + jloc
        p = base + j
        x0, x1 = _threefry2x32(k0, k1, jnp.zeros_like(p), p)
        bits = x0 ^ x1
        fb = (bits >> np.uint32(9)) | np.uint32(0x3F800000)
        f = lax.bitcast_convert_type(fb, jnp.float32) - np.float32(1.0)
        u = jnp.maximum(_TINY, f * (np.float32(1.0) - _TINY) + _TINY)
        g = -jnp.log(-jnp.log(u))
        score = sal / np.float32(_TEMPERATURE) + g
        cm = jnp.max(score)
        cidx = jnp.min(
            jnp.where(score == cm, j.astype(jnp.int32), jnp.int32(0x7FFFFFFF))
        )
        better = cm > m
        m = jnp.where(better, cm, m)
        idx = jnp.where(better, cidx, idx)
        return m, idx

    m, idx = lax.fori_loop(
        0, n_chunks, body, (jnp.float32(-jnp.inf), jnp.int32(0))
    )
    out_ref[0] = jnp.reshape(idx, (1, 1))


def _sample_indices(sal3):
    B, H, W = sal3.shape
    kern = partial(
        _sample_kernel,
        k0=int(_KS_DATA[0]),
        k1=int(_KS_DATA[1]),
        H=H,
        W=W,
        CH=8,
    )
    out = pl.pallas_call(
        kern,
        grid=(B,),
        in_specs=[pl.BlockSpec((1, H, W), lambda b: (b, 0, 0))],
        out_specs=pl.BlockSpec((1, 1, 1), lambda b: (b, 0, 0)),
        out_shape=jax.ShapeDtypeStruct((B, 1, 1), jnp.int32),
    )(sal3)
    return out[:, 0, 0]


def kernel(saliency_map, prev_pos, prev_direction, step, seq_len):
    B, _, H, W = saliency_map.shape
    rk = jax.random.key(42)
    kc1, ks, kr, kc2 = jax.random.split(rk, 4)
    indices = _sample_indices(saliency_map.reshape(B, H, W))
    y = (indices // W).astype(jnp.float32) / max(H - 1, 1)
    x = (indices % W).astype(jnp.float32) / max(W - 1, 1)
    sal_pos = jnp.stack([x, y], axis=-1)
    rand_pos = jax.random.uniform(kr, (B, 2), dtype=jnp.float32)
    explore = jax.random.uniform(kc1, ()) < _EXPLORATION_RATE
    base_pos = jnp.where(explore, rand_pos, sal_pos)
    momentum_pos = jnp.clip(prev_pos + prev_direction * _MAX_STEP_SIZE, 0.0, 1.0)
    use_mom = jax.random.uniform(kc2, ()) > _EXPLORATION_RATE
    mixed = (1.0 - _MOMENTUM) * base_pos + _MOMENTUM * momentum_pos
    base_pos = jnp.where(use_mom, mixed, base_pos)
    return base_pos


# speedup vs baseline: 5.0613x; 5.0613x over previous
import numpy as np
import jax
import jax.numpy as jnp
from jax import lax
from functools import partial
from jax.experimental import pallas as pl
from jax.experimental.pallas import tpu as pltpu

_TEMPERATURE = 0.12
_MAX_STEP_SIZE = 0.18
_MOMENTUM = 0.45
_EXPLORATION_RATE = 0.45

_R1 = (13, 15, 26, 6)
_R2 = (17, 29, 16, 24)
_TINY = np.float32(np.finfo(np.float32).tiny)


def _threefry2x32(k0, k1, x0, x1):
    ks0 = np.uint32(k0)
    ks1 = np.uint32(k1)
    ks2 = np.uint32((int(k0) ^ int(k1) ^ 0x1BD11BDA) & 0xFFFFFFFF)

    def rnds(x0, x1, rots):
        for r in rots:
            x0 = x0 + x1
            x1 = (x1 << np.uint32(r)) | (x1 >> np.uint32(32 - r))
            x1 = x1 ^ x0
        return x0, x1

    x0 = x0 + ks0
    x1 = x1 + ks1
    x0, x1 = rnds(x0, x1, _R1)
    x0 = x0 + ks1
    x1 = x1 + np.uint32((int(ks2) + 1) & 0xFFFFFFFF)
    x0, x1 = rnds(x0, x1, _R2)
    x0 = x0 + ks2
    x1 = x1 + np.uint32((int(ks0) + 2) & 0xFFFFFFFF)
    x0, x1 = rnds(x0, x1, _R1)
    x0 = x0 + ks0
    x1 = x1 + np.uint32((int(ks1) + 3) & 0xFFFFFFFF)
    x0, x1 = rnds(x0, x1, _R2)
    x0 = x0 + ks1
    x1 = x1 + np.uint32((int(ks2) + 4) & 0xFFFFFFFF)
    x0, x1 = rnds(x0, x1, _R1)
    x0 = x0 + ks2
    x1 = x1 + np.uint32((int(ks0) + 5) & 0xFFFFFFFF)
    return x0, x1


def _np_threefry2x32(k0, k1, x0, x1):
    old = np.seterr(over="ignore")
    try:
        out = _threefry2x32(
            np.uint32(k0), np.uint32(k1), np.uint32(x0), np.uint32(x1)
        )
    finally:
        np.seterr(**old)
    return out


_KS_DATA = np.asarray(_np_threefry2x32(0, 42, 0, 1), dtype=np.uint32)


def _sample_kernel(sal_ref, out_ref, *, k0, k1, H, W, CH):
    b = pl.program_id(0)
    n_chunks = H // CH
    base = jnp.uint32(H * W) * jnp.uint32(b)
    jrow = lax.broadcasted_iota(jnp.uint32, (CH, W), 0) * jnp.uint32(W)
    jcol = lax.broadcasted_iota(jnp.uint32, (CH, W), 1)
    jloc = jrow + jcol

    def body(i, carry):
        vmax, vidx = carry
        sal = sal_ref[0, pl.ds(i * CH, CH), :]
        j = jnp.uint32(CH * W) * i.astype(jnp.uint32) + jloc
        p = base + j
        x0, x1 = _threefry2x32(k0, k1, jnp.zeros_like(p), p)
        bits = x0 ^ x1
        fb = (bits >> np.uint32(9)) | np.uint32(0x3F800000)
        f = lax.bitcast_convert_type(fb, jnp.float32) - np.float32(1.0)
        u = jnp.maximum(_TINY, f * (np.float32(1.0) - _TINY) + _TINY)
        g = -jnp.log(-jnp.log(u))
        score = sal / np.float32(_TEMPERATURE) + g
        take = score > vmax
        vmax = jnp.where(take, score, vmax)
        vidx = jnp.where(take, j.astype(jnp.int32), vidx)
        return vmax, vidx

    vmax0 = jnp.full((CH, W), -jnp.inf, dtype=jnp.float32)
    vidx0 = jnp.zeros((CH, W), dtype=jnp.int32)
    vmax, vidx = lax.fori_loop(0, n_chunks, body, (vmax0, vidx0), unroll=2)
    m = jnp.max(vmax)
    idx = jnp.min(jnp.where(vmax == m, vidx, jnp.int32(0x7FFFFFFF)))
    out_ref[0] = jnp.reshape(idx, (1, 1))


def _sample_indices(sal3):
    B, H, W = sal3.shape
    kern = partial(
        _sample_kernel,
        k0=int(_KS_DATA[0]),
        k1=int(_KS_DATA[1]),
        H=H,
        W=W,
        CH=8,
    )
    out = pl.pallas_call(
        kern,
        grid=(B,),
        in_specs=[pl.BlockSpec((1, H, W), lambda b: (b, 0, 0))],
        out_specs=pl.BlockSpec((1, 1, 1), lambda b: (b, 0, 0)),
        out_shape=jax.ShapeDtypeStruct((B, 1, 1), jnp.int32),
    )(sal3)
    return out[:, 0, 0]


def kernel(saliency_map, prev_pos, prev_direction, step, seq_len):
    B, _, H, W = saliency_map.shape
    rk = jax.random.key(42)
    kc1, ks, kr, kc2 = jax.random.split(rk, 4)
    indices = _sample_indices(saliency_map.reshape(B, H, W))
    y = (indices // W).astype(jnp.float32) / max(H - 1, 1)
    x = (indices % W).astype(jnp.float32) / max(W - 1, 1)
    sal_pos = jnp.stack([x, y], axis=-1)
    rand_pos = jax.random.uniform(kr, (B, 2), dtype=jnp.float32)
    explore = jax.random.uniform(kc1, ()) < _EXPLORATION_RATE
    base_pos = jnp.where(explore, rand_pos, sal_pos)
    momentum_pos = jnp.clip(prev_pos + prev_direction * _MAX_STEP_SIZE, 0.0, 1.0)
    use_mom = jax.random.uniform(kc2, ()) > _EXPLORATION_RATE
    mixed = (1.0 - _MOMENTUM) * base_pos + _MOMENTUM * momentum_pos
    base_pos = jnp.where(use_mom, mixed, base_pos)
    return base_pos


# speedup vs baseline: 5.2867x; 1.0445x over previous
import numpy as np
import jax
import jax.numpy as jnp
from jax import lax
from functools import partial
from jax.experimental import pallas as pl
from jax.experimental.pallas import tpu as pltpu

_TEMPERATURE = 0.12
_MAX_STEP_SIZE = 0.18
_MOMENTUM = 0.45
_EXPLORATION_RATE = 0.45

_R1 = (13, 15, 26, 6)
_R2 = (17, 29, 16, 24)
_TINY = np.float32(np.finfo(np.float32).tiny)


def _threefry2x32(k0, k1, x0, x1):
    ks0 = np.uint32(k0)
    ks1 = np.uint32(k1)
    ks2 = np.uint32((int(k0) ^ int(k1) ^ 0x1BD11BDA) & 0xFFFFFFFF)

    def rnds(x0, x1, rots):
        for r in rots:
            x0 = x0 + x1
            x1 = (x1 << np.uint32(r)) | (x1 >> np.uint32(32 - r))
            x1 = x1 ^ x0
        return x0, x1

    x0 = x0 + ks0
    x1 = x1 + ks1
    x0, x1 = rnds(x0, x1, _R1)
    x0 = x0 + ks1
    x1 = x1 + np.uint32((int(ks2) + 1) & 0xFFFFFFFF)
    x0, x1 = rnds(x0, x1, _R2)
    x0 = x0 + ks2
    x1 = x1 + np.uint32((int(ks0) + 2) & 0xFFFFFFFF)
    x0, x1 = rnds(x0, x1, _R1)
    x0 = x0 + ks0
    x1 = x1 + np.uint32((int(ks1) + 3) & 0xFFFFFFFF)
    x0, x1 = rnds(x0, x1, _R2)
    x0 = x0 + ks1
    x1 = x1 + np.uint32((int(ks2) + 4) & 0xFFFFFFFF)
    x0, x1 = rnds(x0, x1, _R1)
    x0 = x0 + ks2
    x1 = x1 + np.uint32((int(ks0) + 5) & 0xFFFFFFFF)
    return x0, x1


def _np_threefry2x32(k0, k1, x0, x1):
    old = np.seterr(over="ignore")
    try:
        out = _threefry2x32(
            np.uint32(k0), np.uint32(k1), np.uint32(x0), np.uint32(x1)
        )
    finally:
        np.seterr(**old)
    return out


_KS_DATA = np.asarray(_np_threefry2x32(0, 42, 0, 1), dtype=np.uint32)


def _sample_kernel(sal_ref, out_ref, idx_ref, *, k0, k1, H, W, CH):
    b = pl.program_id(0)
    n_chunks = H // CH
    base = jnp.uint32(H * W) * jnp.uint32(b)
    jrow = lax.broadcasted_iota(jnp.uint32, (CH, W), 0) * jnp.uint32(W)
    jcol = lax.broadcasted_iota(jnp.uint32, (CH, W), 1)
    jloc = jrow + jcol

    def body(i, carry):
        vmax, vidx = carry
        sal = sal_ref[0, pl.ds(i * CH, CH), :]
        j = jnp.uint32(CH * W) * i.astype(jnp.uint32) + jloc
        p = base + j
        x0, x1 = _threefry2x32(k0, k1, jnp.zeros_like(p), p)
        bits = x0 ^ x1
        fb = (bits >> np.uint32(9)) | np.uint32(0x3F800000)
        f = lax.bitcast_convert_type(fb, jnp.float32) - np.float32(1.0)
        u = jnp.maximum(_TINY, f * (np.float32(1.0) - _TINY) + _TINY)
        g = -jnp.log(-jnp.log(u))
        score = sal / np.float32(_TEMPERATURE) + g
        take = score > vmax
        vmax = jnp.where(take, score, vmax)
        vidx = jnp.where(take, j.astype(jnp.int32), vidx)
        return vmax, vidx

    vmax0 = jnp.full((CH, W), -jnp.inf, dtype=jnp.float32)
    vidx0 = jnp.zeros((CH, W), dtype=jnp.int32)
    vmax, vidx = lax.fori_loop(0, n_chunks, body, (vmax0, vidx0), unroll=2)
    out_ref[0] = vmax
    idx_ref[0] = vidx


def _finalize_kernel(vmax_ref, vidx_ref, pos_ref, *, H, W):
    v = vmax_ref[...]
    vidx = vidx_ref[...]
    m = jnp.max(v, axis=1, keepdims=True)
    cand = jnp.where(v == m, vidx, jnp.int32(0x7FFFFFFF))
    idx = jnp.min(cand, axis=1)
    y = (idx // W).astype(jnp.float32) / np.float32(max(H - 1, 1))
    x = (idx % W).astype(jnp.float32) / np.float32(max(W - 1, 1))
    pos_ref[...] = jnp.concatenate([x[:, None], y[:, None]], axis=1)


def _sample_positions(sal3):
    B, H, W = sal3.shape
    CH = 8
    kern = partial(
        _sample_kernel,
        k0=int(_KS_DATA[0]),
        k1=int(_KS_DATA[1]),
        H=H,
        W=W,
        CH=CH,
    )
    vmax, vidx = pl.pallas_call(
        kern,
        grid=(B,),
        in_specs=[pl.BlockSpec((1, H, W), lambda b: (b, 0, 0))],
        out_specs=[
            pl.BlockSpec((1, CH, W), lambda b: (b, 0, 0)),
            pl.BlockSpec((1, CH, W), lambda b: (b, 0, 0)),
        ],
        out_shape=[
            jax.ShapeDtypeStruct((B, CH, W), jnp.float32),
            jax.ShapeDtypeStruct((B, CH, W), jnp.int32),
        ],
    )(sal3)
    vmax = vmax.reshape(B, CH * W)
    vidx = vidx.reshape(B, CH * W)
    pos = pl.pallas_call(
        partial(_finalize_kernel, H=H, W=W),
        out_shape=jax.ShapeDtypeStruct((B, 2), jnp.float32),
    )(vmax, vidx)
    return pos


def kernel(saliency_map, prev_pos, prev_direction, step, seq_len):
    B, _, H, W = saliency_map.shape
    rk = jax.random.key(42)
    kc1, ks, kr, kc2 = jax.random.split(rk, 4)
    sal_pos = _sample_positions(saliency_map.reshape(B, H, W))
    rand_pos = jax.random.uniform(kr, (B, 2), dtype=jnp.float32)
    explore = jax.random.uniform(kc1, ()) < _EXPLORATION_RATE
    base_pos = jnp.where(explore, rand_pos, sal_pos)
    momentum_pos = jnp.clip(prev_pos + prev_direction * _MAX_STEP_SIZE, 0.0, 1.0)
    use_mom = jax.random.uniform(kc2, ()) > _EXPLORATION_RATE
    mixed = (1.0 - _MOMENTUM) * base_pos + _MOMENTUM * momentum_pos
    base_pos = jnp.where(use_mom, mixed, base_pos)
    return base_pos


# speedup vs baseline: 5.6843x; 1.0752x over previous
import numpy as np
import jax
import jax.numpy as jnp
from jax import lax
from functools import partial
from jax.experimental import pallas as pl
from jax.experimental.pallas import tpu as pltpu

_TEMPERATURE = 0.12
_MAX_STEP_SIZE = 0.18
_MOMENTUM = 0.45
_EXPLORATION_RATE = 0.45

_R1 = (13, 15, 26, 6)
_R2 = (17, 29, 16, 24)
_TINY = np.float32(np.finfo(np.float32).tiny)


def _threefry2x32(k0, k1, x0, x1):
    ks0 = np.uint32(k0)
    ks1 = np.uint32(k1)
    ks2 = np.uint32((int(k0) ^ int(k1) ^ 0x1BD11BDA) & 0xFFFFFFFF)

    def rnds(x0, x1, rots):
        for r in rots:
            x0 = x0 + x1
            x1 = (x1 << np.uint32(r)) | (x1 >> np.uint32(32 - r))
            x1 = x1 ^ x0
        return x0, x1

    x0 = x0 + ks0
    x1 = x1 + ks1
    x0, x1 = rnds(x0, x1, _R1)
    x0 = x0 + ks1
    x1 = x1 + np.uint32((int(ks2) + 1) & 0xFFFFFFFF)
    x0, x1 = rnds(x0, x1, _R2)
    x0 = x0 + ks2
    x1 = x1 + np.uint32((int(ks0) + 2) & 0xFFFFFFFF)
    x0, x1 = rnds(x0, x1, _R1)
    x0 = x0 + ks0
    x1 = x1 + np.uint32((int(ks1) + 3) & 0xFFFFFFFF)
    x0, x1 = rnds(x0, x1, _R2)
    x0 = x0 + ks1
    x1 = x1 + np.uint32((int(ks2) + 4) & 0xFFFFFFFF)
    x0, x1 = rnds(x0, x1, _R1)
    x0 = x0 + ks2
    x1 = x1 + np.uint32((int(ks0) + 5) & 0xFFFFFFFF)
    return x0, x1


def _np_threefry2x32(k0, k1, x0, x1):
    old = np.seterr(over="ignore")
    try:
        out = _threefry2x32(
            np.uint32(k0), np.uint32(k1), np.uint32(x0), np.uint32(x1)
        )
    finally:
        np.seterr(**old)
    return out


_KS_DATA = np.asarray(_np_threefry2x32(0, 42, 0, 1), dtype=np.uint32)


def _sample_kernel(sal_ref, out_ref, idx_ref, *, k0, k1, H, W, CH):
    b = pl.program_id(0)
    n_chunks = H // CH
    base = jnp.uint32(H * W) * jnp.uint32(b)
    jrow = lax.broadcasted_iota(jnp.uint32, (CH, W), 0) * jnp.uint32(W)
    jcol = lax.broadcasted_iota(jnp.uint32, (CH, W), 1)
    jloc = jrow + jcol

    def body(i, carry):
        vmax, vidx = carry
        sal = sal_ref[0, pl.ds(i * CH, CH), :]
        j = jnp.uint32(CH * W) * i.astype(jnp.uint32) + jloc
        p = base + j
        x0, x1 = _threefry2x32(k0, k1, jnp.zeros_like(p), p)
        bits = x0 ^ x1
        fb = (bits >> np.uint32(9)) | np.uint32(0x3F800000)
        f = lax.bitcast_convert_type(fb, jnp.float32) - np.float32(1.0)
        u = jnp.maximum(_TINY, f * (np.float32(1.0) - _TINY) + _TINY)
        g = -jnp.log(-jnp.log(u))
        score = sal / np.float32(_TEMPERATURE) + g
        take = score > vmax
        vmax = jnp.where(take, score, vmax)
        vidx = jnp.where(take, j.astype(jnp.int32), vidx)
        return vmax, vidx

    vmax0 = jnp.full((CH, W), -jnp.inf, dtype=jnp.float32)
    vidx0 = jnp.zeros((CH, W), dtype=jnp.int32)
    vmax, vidx = lax.fori_loop(0, n_chunks, body, (vmax0, vidx0), unroll=4)
    out_ref[0] = vmax
    idx_ref[0] = vidx


def _finalize_kernel(vmax_ref, vidx_ref, pos_ref, *, H, W):
    v = vmax_ref[...]
    vidx = vidx_ref[...]
    m = jnp.max(v, axis=1, keepdims=True)
    cand = jnp.where(v == m, vidx, jnp.int32(0x7FFFFFFF))
    idx = jnp.min(cand, axis=1)
    y = (idx // W).astype(jnp.float32) / np.float32(max(H - 1, 1))
    x = (idx % W).astype(jnp.float32) / np.float32(max(W - 1, 1))
    pos_ref[...] = jnp.concatenate([x[:, None], y[:, None]], axis=1)


def _sample_positions(sal3):
    B, H, W = sal3.shape
    CH = 8
    kern = partial(
        _sample_kernel,
        k0=int(_KS_DATA[0]),
        k1=int(_KS_DATA[1]),
        H=H,
        W=W,
        CH=CH,
    )
    vmax, vidx = pl.pallas_call(
        kern,
        grid=(B,),
        in_specs=[pl.BlockSpec((1, H, W), lambda b: (b, 0, 0))],
        out_specs=[
            pl.BlockSpec((1, CH, W), lambda b: (b, 0, 0)),
            pl.BlockSpec((1, CH, W), lambda b: (b, 0, 0)),
        ],
        out_shape=[
            jax.ShapeDtypeStruct((B, CH, W), jnp.float32),
            jax.ShapeDtypeStruct((B, CH, W), jnp.int32),
        ],
    )(sal3)
    vmax = vmax.reshape(B, CH * W)
    vidx = vidx.reshape(B, CH * W)
    pos = pl.pallas_call(
        partial(_finalize_kernel, H=H, W=W),
        out_shape=jax.ShapeDtypeStruct((B, 2), jnp.float32),
    )(vmax, vidx)
    return pos


def kernel(saliency_map, prev_pos, prev_direction, step, seq_len):
    B, _, H, W = saliency_map.shape
    rk = jax.random.key(42)
    kc1, ks, kr, kc2 = jax.random.split(rk, 4)
    sal_pos = _sample_positions(saliency_map.reshape(B, H, W))
    rand_pos = jax.random.uniform(kr, (B, 2), dtype=jnp.float32)
    explore = jax.random.uniform(kc1, ()) < _EXPLORATION_RATE
    base_pos = jnp.where(explore, rand_pos, sal_pos)
    momentum_pos = jnp.clip(prev_pos + prev_direction * _MAX_STEP_SIZE, 0.0, 1.0)
    use_mom = jax.random.uniform(kc2, ()) > _EXPLORATION_RATE
    mixed = (1.0 - _MOMENTUM) * base_pos + _MOMENTUM * momentum_pos
    base_pos = jnp.where(use_mom, mixed, base_pos)
    return base_pos


# speedup vs baseline: 5.7136x; 1.0052x over previous
import numpy as np
import jax
import jax.numpy as jnp
from jax import lax
from functools import partial
from jax.experimental import pallas as pl
from jax.experimental.pallas import tpu as pltpu

_TEMPERATURE = 0.12
_MAX_STEP_SIZE = 0.18
_MOMENTUM = 0.45
_EXPLORATION_RATE = 0.45

_R1 = (13, 15, 26, 6)
_R2 = (17, 29, 16, 24)
_TINY = np.float32(np.finfo(np.float32).tiny)


def _threefry2x32(k0, k1, x0, x1):
    ks0 = np.uint32(k0)
    ks1 = np.uint32(k1)
    ks2 = np.uint32((int(k0) ^ int(k1) ^ 0x1BD11BDA) & 0xFFFFFFFF)

    def rnds(x0, x1, rots):
        for r in rots:
            x0 = x0 + x1
            x1 = (x1 << np.uint32(r)) | (x1 >> np.uint32(32 - r))
            x1 = x1 ^ x0
        return x0, x1

    x0 = x0 + ks0
    x1 = x1 + ks1
    x0, x1 = rnds(x0, x1, _R1)
    x0 = x0 + ks1
    x1 = x1 + np.uint32((int(ks2) + 1) & 0xFFFFFFFF)
    x0, x1 = rnds(x0, x1, _R2)
    x0 = x0 + ks2
    x1 = x1 + np.uint32((int(ks0) + 2) & 0xFFFFFFFF)
    x0, x1 = rnds(x0, x1, _R1)
    x0 = x0 + ks0
    x1 = x1 + np.uint32((int(ks1) + 3) & 0xFFFFFFFF)
    x0, x1 = rnds(x0, x1, _R2)
    x0 = x0 + ks1
    x1 = x1 + np.uint32((int(ks2) + 4) & 0xFFFFFFFF)
    x0, x1 = rnds(x0, x1, _R1)
    x0 = x0 + ks2
    x1 = x1 + np.uint32((int(ks0) + 5) & 0xFFFFFFFF)
    return x0, x1


def _np_threefry2x32(k0, k1, x0, x1):
    old = np.seterr(over="ignore")
    try:
        out = _threefry2x32(
            np.uint32(k0), np.uint32(k1), np.uint32(x0), np.uint32(x1)
        )
    finally:
        np.seterr(**old)
    return out


_KS_DATA = np.asarray(_np_threefry2x32(0, 42, 0, 1), dtype=np.uint32)


def _sample_kernel(sal_ref, out_ref, idx_ref, *, k0, k1, H, W, CH):
    b = pl.program_id(0)
    n_chunks = H // CH
    base = jnp.uint32(H * W) * jnp.uint32(b)
    jrow = lax.broadcasted_iota(jnp.uint32, (CH, W), 0) * jnp.uint32(W)
    jcol = lax.broadcasted_iota(jnp.uint32, (CH, W), 1)
    jloc = jrow + jcol

    def body(i, carry):
        vmax, vidx = carry
        sal = sal_ref[0, pl.ds(i * CH, CH), :]
        j = jnp.uint32(CH * W) * i.astype(jnp.uint32) + jloc
        p = base + j
        x0, x1 = _threefry2x32(k0, k1, jnp.zeros_like(p), p)
        bits = x0 ^ x1
        fb = (bits >> np.uint32(9)) | np.uint32(0x3F800000)
        f = lax.bitcast_convert_type(fb, jnp.float32) - np.float32(1.0)
        u = jnp.maximum(_TINY, f * (np.float32(1.0) - _TINY) + _TINY)
        g = -jnp.log(-jnp.log(u))
        score = sal / np.float32(_TEMPERATURE) + g
        take = score > vmax
        vmax = jnp.where(take, score, vmax)
        vidx = jnp.where(take, j.astype(jnp.int32), vidx)
        return vmax, vidx

    vmax0 = jnp.full((CH, W), -jnp.inf, dtype=jnp.float32)
    vidx0 = jnp.zeros((CH, W), dtype=jnp.int32)
    vmax, vidx = lax.fori_loop(0, n_chunks, body, (vmax0, vidx0), unroll=2)
    out_ref[0] = vmax
    idx_ref[0] = vidx


def _finalize_kernel(vmax_ref, vidx_ref, pos_ref, *, H, W):
    v = vmax_ref[...]
    vidx = vidx_ref[...]
    m = jnp.max(v, axis=1, keepdims=True)
    cand = jnp.where(v == m, vidx, jnp.int32(0x7FFFFFFF))
    idx = jnp.min(cand, axis=1)
    y = (idx // W).astype(jnp.float32) / np.float32(max(H - 1, 1))
    x = (idx % W).astype(jnp.float32) / np.float32(max(W - 1, 1))
    pos_ref[...] = jnp.concatenate([x[:, None], y[:, None]], axis=1)


def _sample_positions(sal3):
    B, H, W = sal3.shape
    CH = 16
    kern = partial(
        _sample_kernel,
        k0=int(_KS_DATA[0]),
        k1=int(_KS_DATA[1]),
        H=H,
        W=W,
        CH=CH,
    )
    vmax, vidx = pl.pallas_call(
        kern,
        grid=(B,),
        in_specs=[pl.BlockSpec((1, H, W), lambda b: (b, 0, 0))],
        out_specs=[
            pl.BlockSpec((1, CH, W), lambda b: (b, 0, 0)),
            pl.BlockSpec((1, CH, W), lambda b: (b, 0, 0)),
        ],
        out_shape=[
            jax.ShapeDtypeStruct((B, CH, W), jnp.float32),
            jax.ShapeDtypeStruct((B, CH, W), jnp.int32),
        ],
    )(sal3)
    vmax = vmax.reshape(B, CH * W)
    vidx = vidx.reshape(B, CH * W)
    pos = pl.pallas_call(
        partial(_finalize_kernel, H=H, W=W),
        out_shape=jax.ShapeDtypeStruct((B, 2), jnp.float32),
    )(vmax, vidx)
    return pos


def kernel(saliency_map, prev_pos, prev_direction, step, seq_len):
    B, _, H, W = saliency_map.shape
    rk = jax.random.key(42)
    kc1, ks, kr, kc2 = jax.random.split(rk, 4)
    sal_pos = _sample_positions(saliency_map.reshape(B, H, W))
    rand_pos = jax.random.uniform(kr, (B, 2), dtype=jnp.float32)
    explore = jax.random.uniform(kc1, ()) < _EXPLORATION_RATE
    base_pos = jnp.where(explore, rand_pos, sal_pos)
    momentum_pos = jnp.clip(prev_pos + prev_direction * _MAX_STEP_SIZE, 0.0, 1.0)
    use_mom = jax.random.uniform(kc2, ()) > _EXPLORATION_RATE
    mixed = (1.0 - _MOMENTUM) * base_pos + _MOMENTUM * momentum_pos
    base_pos = jnp.where(use_mom, mixed, base_pos)
    return base_pos


# speedup vs baseline: 5.8171x; 1.0181x over previous
import numpy as np
import jax
import jax.numpy as jnp
from jax import lax
from functools import partial
from jax.experimental import pallas as pl
from jax.experimental.pallas import tpu as pltpu

_TEMPERATURE = 0.12
_MAX_STEP_SIZE = 0.18
_MOMENTUM = 0.45
_EXPLORATION_RATE = 0.45

_R1 = (13, 15, 26, 6)
_R2 = (17, 29, 16, 24)
_TINY = np.float32(np.finfo(np.float32).tiny)


def _threefry2x32(k0, k1, x0, x1):
    ks0 = np.uint32(k0)
    ks1 = np.uint32(k1)
    ks2 = np.uint32((int(k0) ^ int(k1) ^ 0x1BD11BDA) & 0xFFFFFFFF)

    def rnds(x0, x1, rots):
        for r in rots:
            x0 = x0 + x1
            x1 = (x1 << np.uint32(r)) | (x1 >> np.uint32(32 - r))
            x1 = x1 ^ x0
        return x0, x1

    x0 = x0 + ks0
    x1 = x1 + ks1
    x0, x1 = rnds(x0, x1, _R1)
    x0 = x0 + ks1
    x1 = x1 + np.uint32((int(ks2) + 1) & 0xFFFFFFFF)
    x0, x1 = rnds(x0, x1, _R2)
    x0 = x0 + ks2
    x1 = x1 + np.uint32((int(ks0) + 2) & 0xFFFFFFFF)
    x0, x1 = rnds(x0, x1, _R1)
    x0 = x0 + ks0
    x1 = x1 + np.uint32((int(ks1) + 3) & 0xFFFFFFFF)
    x0, x1 = rnds(x0, x1, _R2)
    x0 = x0 + ks1
    x1 = x1 + np.uint32((int(ks2) + 4) & 0xFFFFFFFF)
    x0, x1 = rnds(x0, x1, _R1)
    x0 = x0 + ks2
    x1 = x1 + np.uint32((int(ks0) + 5) & 0xFFFFFFFF)
    return x0, x1


def _np_threefry2x32(k0, k1, x0, x1):
    old = np.seterr(over="ignore")
    try:
        out = _threefry2x32(
            np.uint32(k0), np.uint32(k1), np.uint32(x0), np.uint32(x1)
        )
    finally:
        np.seterr(**old)
    return out


_KS_DATA = np.asarray(_np_threefry2x32(0, 42, 0, 1), dtype=np.uint32)


def _sample_kernel(sal_ref, out_ref, idx_ref, *, k0, k1, H, W, CH):
    b = pl.program_id(0)
    n_chunks = H // CH
    base = jnp.uint32(H * W) * jnp.uint32(b)
    jrow = lax.broadcasted_iota(jnp.uint32, (CH, W), 0) * jnp.uint32(W)
    jcol = lax.broadcasted_iota(jnp.uint32, (CH, W), 1)
    jloc = jrow + jcol

    def body(i, carry):
        vmax, vidx = carry
        sal = sal_ref[0, pl.ds(i * CH, CH), :]
        j = jnp.uint32(CH * W) * i.astype(jnp.uint32) + jloc
        p = base + j
        x0, x1 = _threefry2x32(k0, k1, jnp.zeros_like(p), p)
        bits = x0 ^ x1
        fb = (bits >> np.uint32(9)) | np.uint32(0x3F800000)
        f = lax.bitcast_convert_type(fb, jnp.float32) - np.float32(1.0)
        u = jnp.maximum(_TINY, f * (np.float32(1.0) - _TINY) + _TINY)
        g = -jnp.log(-jnp.log(u))
        score = sal / np.float32(_TEMPERATURE) + g
        take = score > vmax
        vmax = jnp.where(take, score, vmax)
        vidx = jnp.where(take, j.astype(jnp.int32), vidx)
        return vmax, vidx

    vmax0 = jnp.full((CH, W), -jnp.inf, dtype=jnp.float32)
    vidx0 = jnp.zeros((CH, W), dtype=jnp.int32)
    vmax, vidx = lax.fori_loop(0, n_chunks, body, (vmax0, vidx0), unroll=4)
    out_ref[0] = vmax
    idx_ref[0] = vidx


def _finalize_kernel(vmax_ref, vidx_ref, pos_ref, *, H, W):
    v = vmax_ref[...]
    vidx = vidx_ref[...]
    m = jnp.max(v, axis=1, keepdims=True)
    cand = jnp.where(v == m, vidx, jnp.int32(0x7FFFFFFF))
    idx = jnp.min(cand, axis=1)
    y = (idx // W).astype(jnp.float32) / np.float32(max(H - 1, 1))
    x = (idx % W).astype(jnp.float32) / np.float32(max(W - 1, 1))
    pos_ref[...] = jnp.concatenate([x[:, None], y[:, None]], axis=1)


def _sample_positions(sal3):
    B, H, W = sal3.shape
    CH = 16
    kern = partial(
        _sample_kernel,
        k0=int(_KS_DATA[0]),
        k1=int(_KS_DATA[1]),
        H=H,
        W=W,
        CH=CH,
    )
    vmax, vidx = pl.pallas_call(
        kern,
        grid=(B,),
        in_specs=[pl.BlockSpec((1, H, W), lambda b: (b, 0, 0))],
        out_specs=[
            pl.BlockSpec((1, CH, W), lambda b: (b, 0, 0)),
            pl.BlockSpec((1, CH, W), lambda b: (b, 0, 0)),
        ],
        out_shape=[
            jax.ShapeDtypeStruct((B, CH, W), jnp.float32),
            jax.ShapeDtypeStruct((B, CH, W), jnp.int32),
        ],
    )(sal3)
    vmax = vmax.reshape(B, CH * W)
    vidx = vidx.reshape(B, CH * W)
    pos = pl.pallas_call(
        partial(_finalize_kernel, H=H, W=W),
        out_shape=jax.ShapeDtypeStruct((B, 2), jnp.float32),
    )(vmax, vidx)
    return pos


def kernel(saliency_map, prev_pos, prev_direction, step, seq_len):
    B, _, H, W = saliency_map.shape
    rk = jax.random.key(42)
    kc1, ks, kr, kc2 = jax.random.split(rk, 4)
    sal_pos = _sample_positions(saliency_map.reshape(B, H, W))
    rand_pos = jax.random.uniform(kr, (B, 2), dtype=jnp.float32)
    explore = jax.random.uniform(kc1, ()) < _EXPLORATION_RATE
    base_pos = jnp.where(explore, rand_pos, sal_pos)
    momentum_pos = jnp.clip(prev_pos + prev_direction * _MAX_STEP_SIZE, 0.0, 1.0)
    use_mom = jax.random.uniform(kc2, ()) > _EXPLORATION_RATE
    mixed = (1.0 - _MOMENTUM) * base_pos + _MOMENTUM * momentum_pos
    base_pos = jnp.where(use_mom, mixed, base_pos)
    return base_pos
